# baseline (speedup 1.0000x reference)
import functools

import jax
import jax.numpy as jnp
from jax import lax
from jax.experimental import pallas as pl
from jax.experimental.pallas import tpu as pltpu
from jax.experimental.pallas import tpu_sc as plsc

BATCH = 16384
D = 64

_NC, _NS = 2, 16
_NW = _NC * _NS
_BPW = BATCH // _NW
_HALF = _BPW // 2


def _sc_gather(tab2u, tab2m, upair, mpair):
    mesh = plsc.VectorSubcoreMesh(core_axis_name="c", subcore_axis_name="s")

    @functools.partial(
        pl.kernel,
        mesh=mesh,
        out_type=[
            jax.ShapeDtypeStruct((BATCH, 2 * D), jnp.float32),
            jax.ShapeDtypeStruct((BATCH, 2 * D), jnp.float32),
        ],
        scratch_types=[
            pltpu.VMEM((_BPW,), jnp.int32),
            pltpu.VMEM((_BPW,), jnp.int32),
            pltpu.VMEM((_HALF, 2 * D), jnp.float32),
            pltpu.VMEM((_HALF, 2 * D), jnp.float32),
            pltpu.SemaphoreType.DMA,
            pltpu.SemaphoreType.DMA,
        ],
        compiler_params=pltpu.CompilerParams(use_tc_tiling_on_sc=True),
    )
    def gather_kernel(utab, mtab, uidx, midx, gu_out, gm_out,
                      uidx_v, midx_v, urows, mrows, semu, semm):
        wid = lax.axis_index("s") * _NC + lax.axis_index("c")
        base = wid * _BPW
        pltpu.sync_copy(uidx.at[pl.ds(base, _BPW)], uidx_v)
        pltpu.sync_copy(midx.at[pl.ds(base, _BPW)], midx_v)
        for h in range(2):
            cu = pltpu.async_copy(
                utab.at[uidx_v.at[pl.ds(h * _HALF, _HALF)]], urows, semu)
            cm = pltpu.async_copy(
                mtab.at[midx_v.at[pl.ds(h * _HALF, _HALF)]], mrows, semm)
            cu.wait()
            cm.wait()
            pltpu.sync_copy(urows, gu_out.at[pl.ds(base + h * _HALF, _HALF)])
            pltpu.sync_copy(mrows, gm_out.at[pl.ds(base + h * _HALF, _HALF)])

    return gather_kernel(tab2u, tab2m, upair, mpair)


def _mlp_body(gu_ref, gm_ref, uid_ref, mid_ref, w1u_ref, w1m_ref,
              b1_ref, g1_ref, be1_ref, w2_ref, b2_ref, g2_ref, be2_ref,
              w3_ref, b3_ref, out_ref):
    upar = uid_ref[...] & 1
    mpar = mid_ref[...] & 1
    gu = gu_ref[...]
    gm = gm_ref[...]
    xu = jnp.where(upar == 1, gu[:, D:], gu[:, :D])
    xm = jnp.where(mpar == 1, gm[:, D:], gm[:, :D])

    h = jnp.dot(xu, w1u_ref[...], preferred_element_type=jnp.float32)
    h = h + jnp.dot(xm, w1m_ref[...], preferred_element_type=jnp.float32)
    h = h + b1_ref[...]
    mu = jnp.mean(h, axis=-1, keepdims=True)
    var = jnp.mean((h - mu) ** 2, axis=-1, keepdims=True)
    h = (h - mu) * lax.rsqrt(var + 1e-5) * g1_ref[...] + be1_ref[...]
    h = jnp.maximum(h, 0.0)

    h = jnp.dot(h, w2_ref[...], preferred_element_type=jnp.float32) + b2_ref[...]
    mu = jnp.mean(h, axis=-1, keepdims=True)
    var = jnp.mean((h - mu) ** 2, axis=-1, keepdims=True)
    h = (h - mu) * lax.rsqrt(var + 1e-5) * g2_ref[...] + be2_ref[...]
    h = jnp.maximum(h, 0.0)

    o = jnp.sum(h * w3_ref[...], axis=-1, keepdims=True) + b3_ref[...]
    out_ref[...] = 5.5 / (1.0 + jnp.exp(-o))


def _tc_mlp(gu, gm, uids2, mids2, W1, b1, g1, be1, W2, b2, g2, be2, W3, b3):
    H1 = W1.shape[1]
    H2 = W2.shape[1]
    BB = 2048
    grid = (BATCH // BB,)

    def xmap(i):
        return (i, 0)

    def wmap(i):
        return (0, 0)

    return pl.pallas_call(
        _mlp_body,
        grid=grid,
        in_specs=[
            pl.BlockSpec((BB, 2 * D), xmap),
            pl.BlockSpec((BB, 2 * D), xmap),
            pl.BlockSpec((BB, 1), xmap),
            pl.BlockSpec((BB, 1), xmap),
            pl.BlockSpec((D, H1), wmap),
            pl.BlockSpec((D, H1), wmap),
            pl.BlockSpec((1, H1), wmap),
            pl.BlockSpec((1, H1), wmap),
            pl.BlockSpec((1, H1), wmap),
            pl.BlockSpec((H1, H2), wmap),
            pl.BlockSpec((1, H2), wmap),
            pl.BlockSpec((1, H2), wmap),
            pl.BlockSpec((1, H2), wmap),
            pl.BlockSpec((1, H2), wmap),
            pl.BlockSpec((1, 1), wmap),
        ],
        out_specs=pl.BlockSpec((BB, 1), xmap),
        out_shape=jax.ShapeDtypeStruct((BATCH, 1), jnp.float32),
    )(
        gu, gm, uids2, mids2,
        W1[:D], W1[D:],
        b1.reshape(1, H1), g1.reshape(1, H1), be1.reshape(1, H1),
        W2,
        b2.reshape(1, H2), g2.reshape(1, H2), be2.reshape(1, H2),
        W3.reshape(1, H2),
        b3.reshape(1, 1),
    )


def kernel(user_ids, movie_ids, user_table, movie_table,
           W1, b1, g1, be1, W2, b2, g2, be2, W3, b3):
    uids = user_ids.astype(jnp.int32)
    mids = movie_ids.astype(jnp.int32)
    tab2u = user_table.reshape(500000, 128)
    tab2m = movie_table.reshape(500000, 128)
    gu, gm = _sc_gather(tab2u, tab2m, uids >> 1, mids >> 1)
    return _tc_mlp(gu, gm, uids.reshape(BATCH, 1), mids.reshape(BATCH, 1),
                   W1, b1, g1, be1, W2, b2, g2, be2, W3, b3)


# speedup vs baseline: 2.2101x; 2.2101x over previous
import functools

import jax
import jax.numpy as jnp
from jax import lax
from jax.experimental import pallas as pl
from jax.experimental.pallas import tpu as pltpu
from jax.experimental.pallas import tpu_sc as plsc

BATCH = 16384
D = 64
_NROWS = 1000000

_NC, _NS = 2, 16
_NW = _NC * _NS
_BPW = BATCH // _NW
_HALF = _BPW // 2

_DBLK = 4096
_OFB = 122
_O = _OFB * _DBLK
_T2ROWS = _NROWS - _O if _NROWS - _O > _O else _O
_T2ROWS = max(_O, _NROWS - _O)
_DGRID = (_T2ROWS + _DBLK - 1) // _DBLK


def _detile_body(ua_ref, ub_ref, ma_ref, mb_ref, t2u_ref, t2m_ref):
    t2u_ref[:, :D] = jnp.transpose(ua_ref[...])
    t2u_ref[:, D:] = jnp.transpose(ub_ref[...])
    t2m_ref[:, :D] = jnp.transpose(ma_ref[...])
    t2m_ref[:, D:] = jnp.transpose(mb_ref[...])


def _tc_detile(ut, mt):
    def amap(j):
        return (0, j)

    def bmap(j):
        return (0, j + _OFB)

    def omap(j):
        return (j, 0)

    return pl.pallas_call(
        _detile_body,
        grid=(_DGRID,),
        in_specs=[
            pl.BlockSpec((D, _DBLK), amap),
            pl.BlockSpec((D, _DBLK), bmap),
            pl.BlockSpec((D, _DBLK), amap),
            pl.BlockSpec((D, _DBLK), bmap),
        ],
        out_specs=[
            pl.BlockSpec((_DBLK, 2 * D), omap),
            pl.BlockSpec((_DBLK, 2 * D), omap),
        ],
        out_shape=[
            jax.ShapeDtypeStruct((_T2ROWS, 2 * D), jnp.float32),
            jax.ShapeDtypeStruct((_T2ROWS, 2 * D), jnp.float32),
        ],
    )(ut, ut, mt, mt)


def _sc_gather(tab2u, tab2m, upair, mpair):
    mesh = plsc.VectorSubcoreMesh(core_axis_name="c", subcore_axis_name="s")

    @functools.partial(
        pl.kernel,
        mesh=mesh,
        out_type=[
            jax.ShapeDtypeStruct((BATCH, 2 * D), jnp.float32),
            jax.ShapeDtypeStruct((BATCH, 2 * D), jnp.float32),
        ],
        scratch_types=[
            pltpu.VMEM((_BPW,), jnp.int32),
            pltpu.VMEM((_BPW,), jnp.int32),
            pltpu.VMEM((_HALF, 2 * D), jnp.float32),
            pltpu.VMEM((_HALF, 2 * D), jnp.float32),
            pltpu.SemaphoreType.DMA,
            pltpu.SemaphoreType.DMA,
        ],
        compiler_params=pltpu.CompilerParams(use_tc_tiling_on_sc=True),
    )
    def gather_kernel(utab, mtab, uidx, midx, gu_out, gm_out,
                      uidx_v, midx_v, urows, mrows, semu, semm):
        wid = lax.axis_index("s") * _NC + lax.axis_index("c")
        base = wid * _BPW
        pltpu.sync_copy(uidx.at[pl.ds(base, _BPW)], uidx_v)
        pltpu.sync_copy(midx.at[pl.ds(base, _BPW)], midx_v)
        for h in range(2):
            cu = pltpu.async_copy(
                utab.at[uidx_v.at[pl.ds(h * _HALF, _HALF)]], urows, semu)
            cm = pltpu.async_copy(
                mtab.at[midx_v.at[pl.ds(h * _HALF, _HALF)]], mrows, semm)
            cu.wait()
            cm.wait()
            pltpu.sync_copy(urows, gu_out.at[pl.ds(base + h * _HALF, _HALF)])
            pltpu.sync_copy(mrows, gm_out.at[pl.ds(base + h * _HALF, _HALF)])

    return gather_kernel(tab2u, tab2m, upair, mpair)


def _mlp_body(gu_ref, gm_ref, uid_ref, mid_ref, w1u_ref, w1m_ref,
              b1_ref, g1_ref, be1_ref, w2_ref, b2_ref, g2_ref, be2_ref,
              w3_ref, b3_ref, out_ref):
    gu = gu_ref[...]
    gm = gm_ref[...]
    xu = jnp.where(uid_ref[...] >= _O, gu[:, D:], gu[:, :D])
    xm = jnp.where(mid_ref[...] >= _O, gm[:, D:], gm[:, :D])

    h = jnp.dot(xu, w1u_ref[...], preferred_element_type=jnp.float32)
    h = h + jnp.dot(xm, w1m_ref[...], preferred_element_type=jnp.float32)
    h = h + b1_ref[...]
    mu = jnp.mean(h, axis=-1, keepdims=True)
    var = jnp.mean((h - mu) ** 2, axis=-1, keepdims=True)
    h = (h - mu) * lax.rsqrt(var + 1e-5) * g1_ref[...] + be1_ref[...]
    h = jnp.maximum(h, 0.0)

    h = jnp.dot(h, w2_ref[...], preferred_element_type=jnp.float32) + b2_ref[...]
    mu = jnp.mean(h, axis=-1, keepdims=True)
    var = jnp.mean((h - mu) ** 2, axis=-1, keepdims=True)
    h = (h - mu) * lax.rsqrt(var + 1e-5) * g2_ref[...] + be2_ref[...]
    h = jnp.maximum(h, 0.0)

    o = jnp.sum(h * w3_ref[...], axis=-1, keepdims=True) + b3_ref[...]
    out_ref[...] = 5.5 / (1.0 + jnp.exp(-o))


def _tc_mlp(gu, gm, uids2, mids2, W1, b1, g1, be1, W2, b2, g2, be2, W3, b3):
    H1 = W1.shape[1]
    H2 = W2.shape[1]
    BB = 2048
    grid = (BATCH // BB,)

    def xmap(i):
        return (i, 0)

    def wmap(i):
        return (0, 0)

    return pl.pallas_call(
        _mlp_body,
        grid=grid,
        in_specs=[
            pl.BlockSpec((BB, 2 * D), xmap),
            pl.BlockSpec((BB, 2 * D), xmap),
            pl.BlockSpec((BB, 1), xmap),
            pl.BlockSpec((BB, 1), xmap),
            pl.BlockSpec((D, H1), wmap),
            pl.BlockSpec((D, H1), wmap),
            pl.BlockSpec((1, H1), wmap),
            pl.BlockSpec((1, H1), wmap),
            pl.BlockSpec((1, H1), wmap),
            pl.BlockSpec((H1, H2), wmap),
            pl.BlockSpec((1, H2), wmap),
            pl.BlockSpec((1, H2), wmap),
            pl.BlockSpec((1, H2), wmap),
            pl.BlockSpec((1, H2), wmap),
            pl.BlockSpec((1, 1), wmap),
        ],
        out_specs=pl.BlockSpec((BB, 1), xmap),
        out_shape=jax.ShapeDtypeStruct((BATCH, 1), jnp.float32),
    )(
        gu, gm, uids2, mids2,
        W1[:D], W1[D:],
        b1.reshape(1, H1), g1.reshape(1, H1), be1.reshape(1, H1),
        W2,
        b2.reshape(1, H2), g2.reshape(1, H2), be2.reshape(1, H2),
        W3.reshape(1, H2),
        b3.reshape(1, 1),
    )


def kernel(user_ids, movie_ids, user_table, movie_table,
           W1, b1, g1, be1, W2, b2, g2, be2, W3, b3):
    uids = user_ids.astype(jnp.int32)
    mids = movie_ids.astype(jnp.int32)
    tab2u, tab2m = _tc_detile(user_table.T, movie_table.T)
    up = jnp.where(uids < _O, uids, uids - _O)
    mp = jnp.where(mids < _O, mids, mids - _O)
    gu, gm = _sc_gather(tab2u, tab2m, up, mp)
    return _tc_mlp(gu, gm, uids.reshape(BATCH, 1), mids.reshape(BATCH, 1),
                   W1, b1, g1, be1, W2, b2, g2, be2, W3, b3)


# speedup vs baseline: 2.3410x; 1.0592x over previous
import functools

import jax
import jax.numpy as jnp
from jax import lax
from jax.experimental import pallas as pl
from jax.experimental.pallas import tpu as pltpu
from jax.experimental.pallas import tpu_sc as plsc

BATCH = 16384
D = 64
_NROWS = 1000000

_NC, _NS = 2, 16
_NW = _NC * _NS
_BPW = BATCH // _NW
_HALF = _BPW // 2

_DBLK = 8192
_OFB = 61
_O = _OFB * _DBLK
_T2ROWS = _NROWS - _O if _NROWS - _O > _O else _O
_T2ROWS = max(_O, _NROWS - _O)
_DGRID = (_T2ROWS + _DBLK - 1) // _DBLK


def _detile_body(ua_ref, ub_ref, ma_ref, mb_ref, t2u_ref, t2m_ref):
    t2u_ref[:, :D] = jnp.transpose(ua_ref[...])
    t2u_ref[:, D:] = jnp.transpose(ub_ref[...])
    t2m_ref[:, :D] = jnp.transpose(ma_ref[...])
    t2m_ref[:, D:] = jnp.transpose(mb_ref[...])


def _tc_detile(ut, mt):
    def amap(j):
        return (0, j)

    def bmap(j):
        return (0, j + _OFB)

    def omap(j):
        return (j, 0)

    return pl.pallas_call(
        _detile_body,
        grid=(_DGRID,),
        in_specs=[
            pl.BlockSpec((D, _DBLK), amap),
            pl.BlockSpec((D, _DBLK), bmap),
            pl.BlockSpec((D, _DBLK), amap),
            pl.BlockSpec((D, _DBLK), bmap),
        ],
        out_specs=[
            pl.BlockSpec((_DBLK, 2 * D), omap),
            pl.BlockSpec((_DBLK, 2 * D), omap),
        ],
        out_shape=[
            jax.ShapeDtypeStruct((_T2ROWS, 2 * D), jnp.float32),
            jax.ShapeDtypeStruct((_T2ROWS, 2 * D), jnp.float32),
        ],
    )(ut, ut, mt, mt)


def _sc_gather(tab2u, tab2m, upair, mpair):
    mesh = plsc.VectorSubcoreMesh(core_axis_name="c", subcore_axis_name="s")

    @functools.partial(
        pl.kernel,
        mesh=mesh,
        out_type=[
            jax.ShapeDtypeStruct((BATCH, 2 * D), jnp.float32),
            jax.ShapeDtypeStruct((BATCH, 2 * D), jnp.float32),
        ],
        scratch_types=[
            pltpu.VMEM((_BPW,), jnp.int32),
            pltpu.VMEM((_BPW,), jnp.int32),
            pltpu.VMEM((_HALF, 2 * D), jnp.float32),
            pltpu.VMEM((_HALF, 2 * D), jnp.float32),
            pltpu.SemaphoreType.DMA,
            pltpu.SemaphoreType.DMA,
        ],
        compiler_params=pltpu.CompilerParams(use_tc_tiling_on_sc=True),
    )
    def gather_kernel(utab, mtab, uidx, midx, gu_out, gm_out,
                      uidx_v, midx_v, urows, mrows, semu, semm):
        wid = lax.axis_index("s") * _NC + lax.axis_index("c")
        base = wid * _BPW
        pltpu.sync_copy(uidx.at[pl.ds(base, _BPW)], uidx_v)
        pltpu.sync_copy(midx.at[pl.ds(base, _BPW)], midx_v)
        for h in range(2):
            cu = pltpu.async_copy(
                utab.at[uidx_v.at[pl.ds(h * _HALF, _HALF)]], urows, semu)
            cm = pltpu.async_copy(
                mtab.at[midx_v.at[pl.ds(h * _HALF, _HALF)]], mrows, semm)
            cu.wait()
            cm.wait()
            pltpu.sync_copy(urows, gu_out.at[pl.ds(base + h * _HALF, _HALF)])
            pltpu.sync_copy(mrows, gm_out.at[pl.ds(base + h * _HALF, _HALF)])

    return gather_kernel(tab2u, tab2m, upair, mpair)


def _mlp_body(gu_ref, gm_ref, uid_ref, mid_ref, w1u_ref, w1m_ref,
              b1_ref, g1_ref, be1_ref, w2_ref, b2_ref, g2_ref, be2_ref,
              w3_ref, b3_ref, out_ref):
    gu = gu_ref[...]
    gm = gm_ref[...]
    xu = jnp.where(uid_ref[...] >= _O, gu[:, D:], gu[:, :D])
    xm = jnp.where(mid_ref[...] >= _O, gm[:, D:], gm[:, :D])

    h = jnp.dot(xu, w1u_ref[...], preferred_element_type=jnp.float32)
    h = h + jnp.dot(xm, w1m_ref[...], preferred_element_type=jnp.float32)
    h = h + b1_ref[...]
    mu = jnp.mean(h, axis=-1, keepdims=True)
    var = jnp.mean((h - mu) ** 2, axis=-1, keepdims=True)
    h = (h - mu) * lax.rsqrt(var + 1e-5) * g1_ref[...] + be1_ref[...]
    h = jnp.maximum(h, 0.0)

    h = jnp.dot(h, w2_ref[...], preferred_element_type=jnp.float32) + b2_ref[...]
    mu = jnp.mean(h, axis=-1, keepdims=True)
    var = jnp.mean((h - mu) ** 2, axis=-1, keepdims=True)
    h = (h - mu) * lax.rsqrt(var + 1e-5) * g2_ref[...] + be2_ref[...]
    h = jnp.maximum(h, 0.0)

    o = jnp.sum(h * w3_ref[...], axis=-1, keepdims=True) + b3_ref[...]
    out_ref[...] = 5.5 / (1.0 + jnp.exp(-o))


def _tc_mlp(gu, gm, uids2, mids2, W1, b1, g1, be1, W2, b2, g2, be2, W3, b3):
    H1 = W1.shape[1]
    H2 = W2.shape[1]
    BB = 2048
    grid = (BATCH // BB,)

    def xmap(i):
        return (i, 0)

    def wmap(i):
        return (0, 0)

    return pl.pallas_call(
        _mlp_body,
        grid=grid,
        in_specs=[
            pl.BlockSpec((BB, 2 * D), xmap),
            pl.BlockSpec((BB, 2 * D), xmap),
            pl.BlockSpec((BB, 1), xmap),
            pl.BlockSpec((BB, 1), xmap),
            pl.BlockSpec((D, H1), wmap),
            pl.BlockSpec((D, H1), wmap),
            pl.BlockSpec((1, H1), wmap),
            pl.BlockSpec((1, H1), wmap),
            pl.BlockSpec((1, H1), wmap),
            pl.BlockSpec((H1, H2), wmap),
            pl.BlockSpec((1, H2), wmap),
            pl.BlockSpec((1, H2), wmap),
            pl.BlockSpec((1, H2), wmap),
            pl.BlockSpec((1, H2), wmap),
            pl.BlockSpec((1, 1), wmap),
        ],
        out_specs=pl.BlockSpec((BB, 1), xmap),
        out_shape=jax.ShapeDtypeStruct((BATCH, 1), jnp.float32),
    )(
        gu, gm, uids2, mids2,
        W1[:D], W1[D:],
        b1.reshape(1, H1), g1.reshape(1, H1), be1.reshape(1, H1),
        W2,
        b2.reshape(1, H2), g2.reshape(1, H2), be2.reshape(1, H2),
        W3.reshape(1, H2),
        b3.reshape(1, 1),
    )


def kernel(user_ids, movie_ids, user_table, movie_table,
           W1, b1, g1, be1, W2, b2, g2, be2, W3, b3):
    uids = user_ids.astype(jnp.int32)
    mids = movie_ids.astype(jnp.int32)
    tab2u, tab2m = _tc_detile(user_table.T, movie_table.T)
    up = jnp.where(uids < _O, uids, uids - _O)
    mp = jnp.where(mids < _O, mids, mids - _O)
    gu, gm = _sc_gather(tab2u, tab2m, up, mp)
    return _tc_mlp(gu, gm, uids.reshape(BATCH, 1), mids.reshape(BATCH, 1),
                   W1, b1, g1, be1, W2, b2, g2, be2, W3, b3)
